# baseline (speedup 1.0000x reference)
import jax
import jax.numpy as jnp
import numpy as np
from jax import lax
from jax.experimental import pallas as pl
from jax.experimental.pallas import tpu as pltpu

_RB = 128


def _i32(v):
    return jnp.int32(np.uint32(v).astype(np.int32))


def _mix_columns(d, seed):
    z = lax.broadcasted_iota(jnp.int32, (1, d), 1) + _i32(seed)
    z = z * _i32(0x85EBCA6B)
    z = z ^ lax.shift_right_logical(z, jnp.int32(13))
    z = z * _i32(0xC2B2AE35)
    z = z ^ lax.shift_right_logical(z, jnp.int32(16))
    return z | jnp.int32(1)


def _fused_body(x_ref, out_ref, flag_ref, h_ref, acc_ref):
    nb = h_ref.shape[0] // 2
    i = pl.program_id(0)

    v = x_ref[...]
    v = jnp.where(v == 0.0, 0.0, v)
    bits = lax.bitcast_convert_type(v, jnp.int32)
    d = bits.shape[1]
    w1 = _mix_columns(d, 0x9E3779B9)
    w2 = _mix_columns(d, 0x7F4A7C15)
    h1 = jnp.sum(bits * w1, axis=1, dtype=jnp.int32).reshape(1, _RB)
    h2 = jnp.sum(bits * w2, axis=1, dtype=jnp.int32).reshape(1, _RB)
    h_ref[pl.ds(i, 1), :] = h1
    h_ref[pl.ds(nb + i, 1), :] = h2
    out_ref[...] = jnp.zeros_like(out_ref)

    a1 = h1.reshape(_RB, 1)
    a2 = h2.reshape(_RB, 1)
    iota_a = lax.broadcasted_iota(jnp.int32, (_RB, _RB), 0)
    iota_b = lax.broadcasted_iota(jnp.int32, (_RB, _RB), 1)
    eqd = (a1 == h1) & (a2 == h2) & (iota_a != iota_b)

    def body(j, s):
        b1 = h_ref[pl.ds(j, 1), :]
        b2 = h_ref[pl.ds(nb + j, 1), :]
        return s | jnp.any((a1 == b1) & (a2 == b2)).astype(jnp.int32)

    dup = lax.fori_loop(0, i, body, jnp.any(eqd).astype(jnp.int32))
    prev = jnp.where(i == 0, 0, acc_ref[0, 0])
    cur = prev | dup
    acc_ref[0, 0] = cur

    @pl.when(i == nb - 1)
    def _emit():
        flag_ref[...] = jnp.zeros((1, 1), jnp.int32) + cur


def _fused_call(x):
    n, d = x.shape
    nb = n // _RB
    return pl.pallas_call(
        _fused_body,
        grid=(nb,),
        in_specs=[
            pl.BlockSpec((_RB, d), lambda i: (i, 0)),
        ],
        out_specs=[
            pl.BlockSpec((_RB, d), lambda i: (i, 0)),
            pl.BlockSpec((1, 1), lambda i: (0, 0)),
        ],
        out_shape=[
            jax.ShapeDtypeStruct((n, d), jnp.float32),
            jax.ShapeDtypeStruct((1, 1), jnp.int32),
        ],
        scratch_shapes=[
            pltpu.VMEM((2 * nb, _RB), jnp.int32),
            pltpu.SMEM((1, 1), jnp.int32),
        ],
    )(x)


def _verify_body(a_ref, b_ref, cnt_ref):
    i = pl.program_id(0)
    j = pl.program_id(1)

    @pl.when((i == 0) & (j == 0))
    def _init():
        cnt_ref[...] = jnp.zeros((1, 1), jnp.int32)

    a = a_ref[...]
    gi = i * _RB + lax.broadcasted_iota(jnp.int32, (_RB,), 0)

    def step(b, acc):
        rowb = b_ref[pl.ds(b, 1), :]
        eq = jnp.all(a == rowb, axis=1)
        offdiag = gi != (j * _RB + b)
        return acc + jnp.sum((eq & offdiag).astype(jnp.int32))

    total = lax.fori_loop(0, _RB, step, jnp.int32(0))
    cnt_ref[...] = cnt_ref[...] + total


def _verify_call(x):
    n, d = x.shape
    nb = n // _RB
    return pl.pallas_call(
        _verify_body,
        grid=(nb, nb),
        in_specs=[
            pl.BlockSpec((_RB, d), lambda i, j: (i, 0)),
            pl.BlockSpec((_RB, d), lambda i, j: (j, 0)),
        ],
        out_specs=pl.BlockSpec((1, 1), lambda i, j: (0, 0)),
        out_shape=jax.ShapeDtypeStruct((1, 1), jnp.int32),
    )(x, x)


def kernel(x):
    zeros, flag = _fused_call(x)
    candidate = flag[0, 0] > 0

    def slow_exact():
        cnt = _verify_call(x)
        return jnp.where(cnt[0, 0] > 0, x, jnp.zeros_like(x))

    return lax.cond(candidate, slow_exact, lambda: zeros)


# speedup vs baseline: 1.4493x; 1.4493x over previous
import functools

import jax
import jax.numpy as jnp
import numpy as np
from jax import lax
from jax.experimental import pallas as pl
from jax.experimental.pallas import tpu as pltpu
from jax.experimental.pallas import tpu_sc as plsc

_RB = 128


def _i32(v):
    return jnp.int32(np.uint32(v).astype(np.int32))


def _mix_columns(d, seed):
    z = lax.broadcasted_iota(jnp.int32, (1, d), 1) + _i32(seed)
    z = z * _i32(0x85EBCA6B)
    z = z ^ lax.shift_right_logical(z, jnp.int32(13))
    z = z * _i32(0xC2B2AE35)
    z = z ^ lax.shift_right_logical(z, jnp.int32(16))
    return z | jnp.int32(1)


def _detect_body(x_ref, flag_ref, h_ref):
    nb = h_ref.shape[0] // 2
    i = pl.program_id(0)

    @pl.when(i < nb)
    def _hash():
        v = x_ref[...]
        v = jnp.where(v == 0.0, 0.0, v)
        bits = lax.bitcast_convert_type(v, jnp.int32)
        d = bits.shape[1]
        w1 = _mix_columns(d, 0x9E3779B9)
        w2 = _mix_columns(d, 0x7F4A7C15)
        h1 = jnp.sum(bits * w1, axis=1, dtype=jnp.int32)
        h2 = jnp.sum(bits * w2, axis=1, dtype=jnp.int32)
        h_ref[pl.ds(i, 1), :] = h1.reshape(1, _RB)
        h_ref[pl.ds(nb + i, 1), :] = h2.reshape(1, _RB)

    @pl.when(i == nb)
    def _pair():
        h1 = h_ref[0:nb, :]
        h2 = h_ref[nb:2 * nb, :]
        h1t = jnp.transpose(h1)
        h2t = jnp.transpose(h2)
        iota_a = lax.broadcasted_iota(jnp.int32, (_RB, _RB), 0)
        iota_b = lax.broadcasted_iota(jnp.int32, (_RB, _RB), 1)
        not_diag = iota_a != iota_b
        acc = jnp.zeros((_RB, _RB), jnp.bool_)
        for bi in range(nb):
            a1 = h1t[:, bi:bi + 1]
            a2 = h2t[:, bi:bi + 1]
            for bj in range(bi, nb):
                b1 = h1[bj:bj + 1, :]
                b2 = h2[bj:bj + 1, :]
                eq = (a1 == b1) & (a2 == b2)
                if bj == bi:
                    eq = eq & not_diag
                acc = acc | eq
        flag_ref[...] = (
            jnp.zeros((1, 1), jnp.int32) + jnp.any(acc).astype(jnp.int32)
        )


def _detect_call(x):
    n, d = x.shape
    nb = n // _RB
    return pl.pallas_call(
        _detect_body,
        grid=(nb + 1,),
        in_specs=[
            pl.BlockSpec((_RB, d), lambda i: (jnp.minimum(i, nb - 1), 0)),
        ],
        out_specs=[
            pl.BlockSpec((1, 1), lambda i: (0, 0)),
        ],
        out_shape=[
            jax.ShapeDtypeStruct((1, 1), jnp.int32),
        ],
        scratch_shapes=[pltpu.VMEM((2 * nb, _RB), jnp.int32)],
    )(x)


_NW = 32


def _sc_zeros_call(n, d):
    rpw = n // _NW
    br = rpw if rpw < 16 else 16
    ncopy = rpw // br
    mesh = plsc.VectorSubcoreMesh(core_axis_name="c", subcore_axis_name="s")

    @functools.partial(
        pl.kernel,
        mesh=mesh,
        out_type=jax.ShapeDtypeStruct((n, d), jnp.float32),
        scratch_types=[pltpu.VMEM((br, d), jnp.float32)],
    )
    def k(out_hbm, zbuf):
        z16 = jnp.zeros((16,), jnp.float32)
        for r in range(br):
            def zrow(c, carry):
                zbuf[r, pl.ds(c * 16, 16)] = z16
                return carry
            lax.fori_loop(0, d // 16, zrow, 0)
        wid = lax.axis_index("s") * 2 + lax.axis_index("c")
        base = wid * rpw
        for t in range(ncopy):
            pltpu.sync_copy(zbuf, out_hbm.at[pl.ds(base + t * br, br)])

    return k()


def _verify_body(a_ref, b_ref, cnt_ref):
    i = pl.program_id(0)
    j = pl.program_id(1)

    @pl.when((i == 0) & (j == 0))
    def _init():
        cnt_ref[...] = jnp.zeros((1, 1), jnp.int32)

    a = a_ref[...]
    gi = i * _RB + lax.broadcasted_iota(jnp.int32, (_RB,), 0)

    def step(b, acc):
        rowb = b_ref[pl.ds(b, 1), :]
        eq = jnp.all(a == rowb, axis=1)
        offdiag = gi != (j * _RB + b)
        return acc + jnp.sum((eq & offdiag).astype(jnp.int32))

    total = lax.fori_loop(0, _RB, step, jnp.int32(0))
    cnt_ref[...] = cnt_ref[...] + total


def _verify_call(x):
    n, d = x.shape
    nb = n // _RB
    return pl.pallas_call(
        _verify_body,
        grid=(nb, nb),
        in_specs=[
            pl.BlockSpec((_RB, d), lambda i, j: (i, 0)),
            pl.BlockSpec((_RB, d), lambda i, j: (j, 0)),
        ],
        out_specs=pl.BlockSpec((1, 1), lambda i, j: (0, 0)),
        out_shape=jax.ShapeDtypeStruct((1, 1), jnp.int32),
    )(x, x)


def kernel(x):
    n, d = x.shape
    zeros = _sc_zeros_call(n, d)
    (flag,) = _detect_call(x)
    candidate = flag[0, 0] > 0

    def slow_exact():
        cnt = _verify_call(x)
        return jnp.where(cnt[0, 0] > 0, x, jnp.zeros_like(x))

    return lax.cond(candidate, slow_exact, lambda: zeros)


# speedup vs baseline: 1.8527x; 1.2783x over previous
import jax
import jax.numpy as jnp
import numpy as np
from jax import lax
from jax.experimental import pallas as pl
from jax.experimental.pallas import tpu as pltpu

_RB = 128


def _i32(v):
    return jnp.int32(np.uint32(v).astype(np.int32))


def _mix_columns(d, seed):
    z = lax.broadcasted_iota(jnp.int32, (1, d), 1) + _i32(seed)
    z = z * _i32(0x85EBCA6B)
    z = z ^ lax.shift_right_logical(z, jnp.int32(13))
    z = z * _i32(0xC2B2AE35)
    z = z ^ lax.shift_right_logical(z, jnp.int32(16))
    return z | jnp.int32(1)


def _detect_body(x_ref, out_ref, flag_ref, h_ref):
    nb = h_ref.shape[0] // 2
    i = pl.program_id(0)

    @pl.when(i < nb)
    def _hash():
        v = x_ref[...]
        v = jnp.where(v == 0.0, 0.0, v)
        bits = lax.bitcast_convert_type(v, jnp.int32)
        d = bits.shape[1]
        w1 = _mix_columns(d, 0x9E3779B9)
        w2 = _mix_columns(d, 0x7F4A7C15)
        h1 = jnp.sum(bits * w1, axis=1, dtype=jnp.int32)
        h2 = jnp.sum(bits * w2, axis=1, dtype=jnp.int32)
        h_ref[pl.ds(i, 1), :] = h1.reshape(1, _RB)
        h_ref[pl.ds(nb + i, 1), :] = h2.reshape(1, _RB)
        out_ref[...] = jnp.zeros_like(out_ref)

    @pl.when(i == nb)
    def _pair():
        h1 = h_ref[0:nb, :]
        h2 = h_ref[nb:2 * nb, :]
        h1t = jnp.transpose(h1)
        h2t = jnp.transpose(h2)
        iota_a = lax.broadcasted_iota(jnp.int32, (_RB, _RB), 0)
        iota_b = lax.broadcasted_iota(jnp.int32, (_RB, _RB), 1)
        not_diag = iota_a != iota_b
        acc = jnp.zeros((_RB, _RB), jnp.bool_)
        for bi in range(nb):
            a1 = h1t[:, bi:bi + 1]
            a2 = h2t[:, bi:bi + 1]
            for bj in range(bi, nb):
                b1 = h1[bj:bj + 1, :]
                b2 = h2[bj:bj + 1, :]
                eq = (a1 == b1) & (a2 == b2)
                if bj == bi:
                    eq = eq & not_diag
                acc = acc | eq
        flag_ref[...] = (
            jnp.zeros((1, 1), jnp.int32) + jnp.any(acc).astype(jnp.int32)
        )


def _detect_call(x):
    n, d = x.shape
    nb = n // _RB
    return pl.pallas_call(
        _detect_body,
        grid=(nb + 1,),
        in_specs=[
            pl.BlockSpec((_RB, d), lambda i: (jnp.minimum(i, nb - 1), 0)),
        ],
        out_specs=[
            pl.BlockSpec((_RB, d), lambda i: (jnp.minimum(i, nb - 1), 0)),
            pl.BlockSpec((1, 1), lambda i: (0, 0)),
        ],
        out_shape=[
            jax.ShapeDtypeStruct((n, d), jnp.float32),
            jax.ShapeDtypeStruct((1, 1), jnp.int32),
        ],
        scratch_shapes=[pltpu.VMEM((2 * nb, _RB), jnp.int32)],
    )(x)


def _verify_body(a_ref, b_ref, cnt_ref):
    i = pl.program_id(0)
    j = pl.program_id(1)

    @pl.when((i == 0) & (j == 0))
    def _init():
        cnt_ref[...] = jnp.zeros((1, 1), jnp.int32)

    a = a_ref[...]
    gi = i * _RB + lax.broadcasted_iota(jnp.int32, (_RB,), 0)

    def step(b, acc):
        rowb = b_ref[pl.ds(b, 1), :]
        eq = jnp.all(a == rowb, axis=1)
        offdiag = gi != (j * _RB + b)
        return acc + jnp.sum((eq & offdiag).astype(jnp.int32))

    total = lax.fori_loop(0, _RB, step, jnp.int32(0))
    cnt_ref[...] = cnt_ref[...] + total


def _verify_call(x):
    n, d = x.shape
    nb = n // _RB
    return pl.pallas_call(
        _verify_body,
        grid=(nb, nb),
        in_specs=[
            pl.BlockSpec((_RB, d), lambda i, j: (i, 0)),
            pl.BlockSpec((_RB, d), lambda i, j: (j, 0)),
        ],
        out_specs=pl.BlockSpec((1, 1), lambda i, j: (0, 0)),
        out_shape=jax.ShapeDtypeStruct((1, 1), jnp.int32),
    )(x, x)


def kernel(x):
    zeros, flag = _detect_call(x)
    candidate = flag[0, 0] > 0

    def slow_exact():
        cnt = _verify_call(x)
        return jnp.where(cnt[0, 0] > 0, x, jnp.zeros_like(x))

    return lax.cond(candidate, slow_exact, lambda: zeros)


# speedup vs baseline: 2.3094x; 1.2465x over previous
import jax
import jax.numpy as jnp
import numpy as np
from jax import lax
from jax.experimental import pallas as pl
from jax.experimental.pallas import tpu as pltpu

_RB = 128


def _i32(v):
    return jnp.int32(np.uint32(v).astype(np.int32))


def _mix_columns(d, seed):
    z = lax.broadcasted_iota(jnp.int32, (1, d), 1) + _i32(seed)
    z = z * _i32(0x85EBCA6B)
    z = z ^ lax.shift_right_logical(z, jnp.int32(13))
    z = z * _i32(0xC2B2AE35)
    z = z ^ lax.shift_right_logical(z, jnp.int32(16))
    return z | jnp.int32(1)


def _detect_body(x_ref, out_ref, flag_ref, h_ref):
    nb = h_ref.shape[0] // 2
    rbh = x_ref.shape[0]
    rpb = rbh // _RB
    i = pl.program_id(0)
    nh = nb // rpb

    @pl.when(i < nh)
    def _hash():
        v = x_ref[...]
        v = jnp.where(v == 0.0, 0.0, v)
        bits = lax.bitcast_convert_type(v, jnp.int32)
        d = bits.shape[1]
        w1 = _mix_columns(d, 0x9E3779B9)
        w2 = _mix_columns(d, 0x7F4A7C15)
        h1 = jnp.sum(bits * w1, axis=1, dtype=jnp.int32)
        h2 = jnp.sum(bits * w2, axis=1, dtype=jnp.int32)
        h_ref[pl.ds(i * rpb, rpb), :] = h1.reshape(rpb, _RB)
        h_ref[pl.ds(nb + i * rpb, rpb), :] = h2.reshape(rpb, _RB)
        out_ref[...] = jnp.zeros_like(out_ref)

    @pl.when(i == nh)
    def _pair():
        h1 = h_ref[0:nb, :]
        h2 = h_ref[nb:2 * nb, :]
        h1t = jnp.transpose(h1)
        h2t = jnp.transpose(h2)
        iota_a = lax.broadcasted_iota(jnp.int32, (_RB, _RB), 0)
        iota_b = lax.broadcasted_iota(jnp.int32, (_RB, _RB), 1)
        not_diag = iota_a != iota_b
        acc = jnp.zeros((_RB, _RB), jnp.bool_)
        for bi in range(nb):
            a1 = h1t[:, bi:bi + 1]
            a2 = h2t[:, bi:bi + 1]
            for bj in range(bi, nb):
                b1 = h1[bj:bj + 1, :]
                b2 = h2[bj:bj + 1, :]
                eq = (a1 == b1) & (a2 == b2)
                if bj == bi:
                    eq = eq & not_diag
                acc = acc | eq
        flag_ref[...] = (
            jnp.zeros((1, 1), jnp.int32) + jnp.any(acc).astype(jnp.int32)
        )


def _detect_call(x):
    n, d = x.shape
    nb = n // _RB
    rbh = 256 if n % 256 == 0 else _RB
    nh = n // rbh
    return pl.pallas_call(
        _detect_body,
        grid=(nh + 1,),
        in_specs=[
            pl.BlockSpec((rbh, d), lambda i: (jnp.minimum(i, nh - 1), 0)),
        ],
        out_specs=[
            pl.BlockSpec((rbh, d), lambda i: (jnp.minimum(i, nh - 1), 0)),
            pl.BlockSpec((1, 1), lambda i: (0, 0)),
        ],
        out_shape=[
            jax.ShapeDtypeStruct((n, d), jnp.float32),
            jax.ShapeDtypeStruct((1, 1), jnp.int32),
        ],
        scratch_shapes=[pltpu.VMEM((2 * nb, _RB), jnp.int32)],
    )(x)


def _verify_body(a_ref, b_ref, cnt_ref):
    i = pl.program_id(0)
    j = pl.program_id(1)

    @pl.when((i == 0) & (j == 0))
    def _init():
        cnt_ref[...] = jnp.zeros((1, 1), jnp.int32)

    a = a_ref[...]
    gi = i * _RB + lax.broadcasted_iota(jnp.int32, (_RB,), 0)

    def step(b, acc):
        rowb = b_ref[pl.ds(b, 1), :]
        eq = jnp.all(a == rowb, axis=1)
        offdiag = gi != (j * _RB + b)
        return acc + jnp.sum((eq & offdiag).astype(jnp.int32))

    total = lax.fori_loop(0, _RB, step, jnp.int32(0))
    cnt_ref[...] = cnt_ref[...] + total


def _verify_call(x):
    n, d = x.shape
    nb = n // _RB
    return pl.pallas_call(
        _verify_body,
        grid=(nb, nb),
        in_specs=[
            pl.BlockSpec((_RB, d), lambda i, j: (i, 0)),
            pl.BlockSpec((_RB, d), lambda i, j: (j, 0)),
        ],
        out_specs=pl.BlockSpec((1, 1), lambda i, j: (0, 0)),
        out_shape=jax.ShapeDtypeStruct((1, 1), jnp.int32),
    )(x, x)


def kernel(x):
    zeros, flag = _detect_call(x)
    candidate = flag[0, 0] > 0

    def slow_exact():
        cnt = _verify_call(x)
        return jnp.where(cnt[0, 0] > 0, x, jnp.zeros_like(x))

    return lax.cond(candidate, slow_exact, lambda: zeros)


# speedup vs baseline: 2.5089x; 1.0864x over previous
import jax
import jax.numpy as jnp
import numpy as np
from jax import lax
from jax.experimental import pallas as pl
from jax.experimental.pallas import tpu as pltpu

_RB = 128


def _i32(v):
    return jnp.int32(np.uint32(v).astype(np.int32))


def _mix_columns(d, seed):
    z = lax.broadcasted_iota(jnp.int32, (1, d), 1) + _i32(seed)
    z = z * _i32(0x85EBCA6B)
    z = z ^ lax.shift_right_logical(z, jnp.int32(13))
    z = z * _i32(0xC2B2AE35)
    z = z ^ lax.shift_right_logical(z, jnp.int32(16))
    return z | jnp.int32(1)


def _detect_body(x_ref, out_ref, flag_ref, h_ref):
    nb = h_ref.shape[0] // 2
    rbh = x_ref.shape[0]
    rpb = rbh // _RB
    i = pl.program_id(0)
    nh = nb // rpb

    @pl.when(i < nh)
    def _hash():
        v = x_ref[...]
        v = jnp.where(v == 0.0, 0.0, v)
        bits = lax.bitcast_convert_type(v, jnp.int32)
        d = bits.shape[1]
        w1 = _mix_columns(d, 0x9E3779B9)
        w2 = _mix_columns(d, 0x7F4A7C15)
        h1 = jnp.sum(bits * w1, axis=1, dtype=jnp.int32)
        h2 = jnp.sum(bits * w2, axis=1, dtype=jnp.int32)
        h_ref[pl.ds(i * rpb, rpb), :] = h1.reshape(rpb, _RB)
        h_ref[pl.ds(nb + i * rpb, rpb), :] = h2.reshape(rpb, _RB)
        out_ref[...] = jnp.zeros_like(out_ref)

    @pl.when(i == nh)
    def _pair():
        h1 = h_ref[0:nb, :]
        h2 = h_ref[nb:2 * nb, :]
        h1t = jnp.transpose(h1)
        h2t = jnp.transpose(h2)
        iota_a = lax.broadcasted_iota(jnp.int32, (_RB, _RB), 0)
        iota_b = lax.broadcasted_iota(jnp.int32, (_RB, _RB), 1)
        not_diag = iota_a != iota_b
        acc = jnp.zeros((_RB, _RB), jnp.bool_)
        for bi in range(nb):
            a1 = h1t[:, bi:bi + 1]
            a2 = h2t[:, bi:bi + 1]
            for bj in range(bi, nb):
                b1 = h1[bj:bj + 1, :]
                b2 = h2[bj:bj + 1, :]
                eq = (a1 == b1) & (a2 == b2)
                if bj == bi:
                    eq = eq & not_diag
                acc = acc | eq
        flag_ref[...] = (
            jnp.zeros((1, 1), jnp.int32) + jnp.any(acc).astype(jnp.int32)
        )


def _detect_call(x):
    n, d = x.shape
    nb = n // _RB
    rbh = 512 if n % 512 == 0 else _RB
    nh = n // rbh
    return pl.pallas_call(
        _detect_body,
        grid=(nh + 1,),
        in_specs=[
            pl.BlockSpec((rbh, d), lambda i: (jnp.minimum(i, nh - 1), 0)),
        ],
        out_specs=[
            pl.BlockSpec((rbh, d), lambda i: (jnp.minimum(i, nh - 1), 0)),
            pl.BlockSpec((1, 1), lambda i: (0, 0)),
        ],
        out_shape=[
            jax.ShapeDtypeStruct((n, d), jnp.float32),
            jax.ShapeDtypeStruct((1, 1), jnp.int32),
        ],
        scratch_shapes=[pltpu.VMEM((2 * nb, _RB), jnp.int32)],
    )(x)


def _verify_body(a_ref, b_ref, cnt_ref):
    i = pl.program_id(0)
    j = pl.program_id(1)

    @pl.when((i == 0) & (j == 0))
    def _init():
        cnt_ref[...] = jnp.zeros((1, 1), jnp.int32)

    a = a_ref[...]
    gi = i * _RB + lax.broadcasted_iota(jnp.int32, (_RB,), 0)

    def step(b, acc):
        rowb = b_ref[pl.ds(b, 1), :]
        eq = jnp.all(a == rowb, axis=1)
        offdiag = gi != (j * _RB + b)
        return acc + jnp.sum((eq & offdiag).astype(jnp.int32))

    total = lax.fori_loop(0, _RB, step, jnp.int32(0))
    cnt_ref[...] = cnt_ref[...] + total


def _verify_call(x):
    n, d = x.shape
    nb = n // _RB
    return pl.pallas_call(
        _verify_body,
        grid=(nb, nb),
        in_specs=[
            pl.BlockSpec((_RB, d), lambda i, j: (i, 0)),
            pl.BlockSpec((_RB, d), lambda i, j: (j, 0)),
        ],
        out_specs=pl.BlockSpec((1, 1), lambda i, j: (0, 0)),
        out_shape=jax.ShapeDtypeStruct((1, 1), jnp.int32),
    )(x, x)


def kernel(x):
    zeros, flag = _detect_call(x)
    candidate = flag[0, 0] > 0

    def slow_exact():
        cnt = _verify_call(x)
        return jnp.where(cnt[0, 0] > 0, x, jnp.zeros_like(x))

    return lax.cond(candidate, slow_exact, lambda: zeros)


# speedup vs baseline: 2.5322x; 1.0093x over previous
import jax
import jax.numpy as jnp
import numpy as np
from jax import lax
from jax.experimental import pallas as pl
from jax.experimental.pallas import tpu as pltpu

_RB = 128


def _i32(v):
    return jnp.int32(np.uint32(v).astype(np.int32))


def _mix_columns(d, seed):
    z = lax.broadcasted_iota(jnp.int32, (1, d), 1) + _i32(seed)
    z = z * _i32(0x85EBCA6B)
    z = z ^ lax.shift_right_logical(z, jnp.int32(13))
    z = z * _i32(0xC2B2AE35)
    z = z ^ lax.shift_right_logical(z, jnp.int32(16))
    return z | jnp.int32(1)


def _detect_body(x_ref, out_ref, flag_ref, h_ref):
    nb = h_ref.shape[0] // 2
    rbh = x_ref.shape[0]
    rpb = rbh // _RB
    i = pl.program_id(0)
    nh = nb // rpb

    @pl.when(i < nh)
    def _hash():
        v = x_ref[...]
        v = jnp.where(v == 0.0, 0.0, v)
        bits = lax.bitcast_convert_type(v, jnp.int32)
        d = bits.shape[1]
        w1 = _mix_columns(d, 0x9E3779B9)
        w2 = _mix_columns(d, 0x7F4A7C15)
        h1 = jnp.sum(bits * w1, axis=1, dtype=jnp.int32)
        h2 = jnp.sum(bits * w2, axis=1, dtype=jnp.int32)
        h_ref[pl.ds(i * rpb, rpb), :] = h1.reshape(rpb, _RB)
        h_ref[pl.ds(nb + i * rpb, rpb), :] = h2.reshape(rpb, _RB)
        out_ref[...] = jnp.zeros_like(out_ref)

    @pl.when(i == nh)
    def _pair():
        h1 = h_ref[0:nb, :]
        h2 = h_ref[nb:2 * nb, :]
        h1t = jnp.transpose(h1)
        h2t = jnp.transpose(h2)
        iota_a = lax.broadcasted_iota(jnp.int32, (_RB, _RB), 0)
        iota_b = lax.broadcasted_iota(jnp.int32, (_RB, _RB), 1)
        not_diag = iota_a != iota_b
        acc = jnp.zeros((_RB, _RB), jnp.bool_)
        for bi in range(nb):
            a1 = h1t[:, bi:bi + 1]
            a2 = h2t[:, bi:bi + 1]
            for bj in range(bi, nb):
                b1 = h1[bj:bj + 1, :]
                b2 = h2[bj:bj + 1, :]
                eq = (a1 == b1) & (a2 == b2)
                if bj == bi:
                    eq = eq & not_diag
                acc = acc | eq
        flag_ref[...] = (
            jnp.zeros((1, 1), jnp.int32) + jnp.any(acc).astype(jnp.int32)
        )


def _detect_call(x):
    n, d = x.shape
    nb = n // _RB
    rbh = 1024 if n % 1024 == 0 else _RB
    nh = n // rbh
    return pl.pallas_call(
        _detect_body,
        grid=(nh + 1,),
        in_specs=[
            pl.BlockSpec((rbh, d), lambda i: (jnp.minimum(i, nh - 1), 0)),
        ],
        out_specs=[
            pl.BlockSpec((rbh, d), lambda i: (jnp.minimum(i, nh - 1), 0)),
            pl.BlockSpec((1, 1), lambda i: (0, 0)),
        ],
        out_shape=[
            jax.ShapeDtypeStruct((n, d), jnp.float32),
            jax.ShapeDtypeStruct((1, 1), jnp.int32),
        ],
        scratch_shapes=[pltpu.VMEM((2 * nb, _RB), jnp.int32)],
    )(x)


def _verify_body(a_ref, b_ref, cnt_ref):
    i = pl.program_id(0)
    j = pl.program_id(1)

    @pl.when((i == 0) & (j == 0))
    def _init():
        cnt_ref[...] = jnp.zeros((1, 1), jnp.int32)

    a = a_ref[...]
    gi = i * _RB + lax.broadcasted_iota(jnp.int32, (_RB,), 0)

    def step(b, acc):
        rowb = b_ref[pl.ds(b, 1), :]
        eq = jnp.all(a == rowb, axis=1)
        offdiag = gi != (j * _RB + b)
        return acc + jnp.sum((eq & offdiag).astype(jnp.int32))

    total = lax.fori_loop(0, _RB, step, jnp.int32(0))
    cnt_ref[...] = cnt_ref[...] + total


def _verify_call(x):
    n, d = x.shape
    nb = n // _RB
    return pl.pallas_call(
        _verify_body,
        grid=(nb, nb),
        in_specs=[
            pl.BlockSpec((_RB, d), lambda i, j: (i, 0)),
            pl.BlockSpec((_RB, d), lambda i, j: (j, 0)),
        ],
        out_specs=pl.BlockSpec((1, 1), lambda i, j: (0, 0)),
        out_shape=jax.ShapeDtypeStruct((1, 1), jnp.int32),
    )(x, x)


def kernel(x):
    zeros, flag = _detect_call(x)
    candidate = flag[0, 0] > 0

    def slow_exact():
        cnt = _verify_call(x)
        return jnp.where(cnt[0, 0] > 0, x, jnp.zeros_like(x))

    return lax.cond(candidate, slow_exact, lambda: zeros)
